# baseline (speedup 1.0000x reference)
import functools

import jax
import jax.numpy as jnp
from jax import lax
from jax.experimental import pallas as pl
from jax.experimental.pallas import tpu as pltpu
from jax.experimental.pallas import tpu_sc as plsc

_N = 10000
_E = 320000
_DIN = 128
_DIM = 64
_DOUT = 10
_G = 64

_NC = 2
_NS = 16
_NW = _NC * _NS
_NPAD = 10016
_RPS = _NPAD // _NS
_K = 128
_EPAD = 327680
_EPW = _EPAD // _NW
_CHUNKS = _EPW // _K
_PAT = "SSSS"

_PERM = tuple(
    32 * j + (m // 2 if m % 2 == 0 else 16 + (m - 1) // 2)
    for j in (0, 1) for m in range(32)
)


def _segsum_sc(table, table_bf, src, dst, zeros_blk):
  mesh = plsc.VectorSubcoreMesh(core_axis_name="c", subcore_axis_name="s")

  @functools.partial(
      pl.kernel,
      out_type=jax.ShapeDtypeStruct((_NC, _NPAD, _DIM), jnp.float32),
      mesh=mesh,
      scratch_types=[
          pltpu.VMEM((_CHUNKS, _K), jnp.int32),
          pltpu.VMEM((_CHUNKS, _K), jnp.int32),
          [pltpu.VMEM((_K, _DIM),
                      jnp.float32 if p == "H" else jnp.bfloat16)
           for p in _PAT],
          [pltpu.VMEM((_K, _DIM), jnp.float32) for _ in _PAT],
          pltpu.VMEM_SHARED((_NPAD, _DIM), jnp.float32),
          pltpu.VMEM_SHARED((_N, _DIM), jnp.bfloat16),
          [pltpu.SemaphoreType.DMA for _ in _PAT],
      ],
      compiler_params=pltpu.CompilerParams(use_tc_tiling_on_sc=False,
                                           needs_layout_passes=False),
  )
  def k(table_hbm, tbf_hbm, src_hbm, dst_hbm, z_hbm, out_hbm, sidx, didx,
        rows, conv, accum, tbl, sems):
    c = lax.axis_index("c")
    s = lax.axis_index("s")
    wid = c * _NS + s

    pltpu.sync_copy(z_hbm, accum.at[pl.ds(s * _RPS, _RPS)])
    pltpu.sync_copy(tbf_hbm.at[pl.ds(s * (_N // _NS), _N // _NS)],
                    tbl.at[pl.ds(s * (_N // _NS), _N // _NS)])

    pltpu.sync_copy(src_hbm.at[pl.ds(wid * _CHUNKS, _CHUNKS)], sidx)
    pltpu.sync_copy(dst_hbm.at[pl.ds(wid * _CHUNKS, _CHUNKS)], didx)
    plsc.subcore_barrier()

    nslot = len(_PAT)

    def gather(i, b):
      tab = table_hbm if _PAT[b] == "H" else tbl
      return pltpu.make_async_copy(tab.at[sidx.at[i]], rows[b], sems[b])

    def scatter_add(i, buf):
      pltpu.sync_copy(buf, accum.at[didx.at[i]], add=True)

    def convert(b):
      @pl.loop(0, _K)
      def _(r):
        for j in (0, 1):
          v = rows[b][r, pl.ds(32 * j, 32)]
          w = plsc.bitcast(v, jnp.int32)
          lo = plsc.bitcast(jnp.left_shift(w, 16), jnp.float32)
          hi = plsc.bitcast(jnp.bitwise_and(w, jnp.int32(-65536)),
                            jnp.float32)
          conv[b][r, pl.ds(32 * j, 16)] = lo
          conv[b][r, pl.ds(32 * j + 16, 16)] = hi

    for b in range(nslot):
      gather(b, b).start()

    @pl.loop(0, _CHUNKS, step=nslot)
    def _(i):
      for b in range(nslot):
        ic = i + b
        gather(ic, b).wait()
        if _PAT[b] == "S":
          convert(b)

          @pl.when(ic + nslot < _CHUNKS)
          def _():
            gather(ic + nslot, b).start()

          scatter_add(ic, conv[b])
        else:
          scatter_add(ic, rows[b])

          @pl.when(ic + nslot < _CHUNKS)
          def _():
            gather(ic + nslot, b).start()

    plsc.subcore_barrier()
    pltpu.sync_copy(accum.at[pl.ds(s * _RPS, _RPS)],
                    out_hbm.at[c, pl.ds(s * _RPS, _RPS)])

  return k(table, table_bf, src, dst, zeros_blk)


def _proj_body(x_ref, w_ref, o_ref):
  o_ref[...] = jnp.dot(x_ref[...], w_ref[...],
                       preferred_element_type=jnp.float32)


def _mid_body(agg_ref, y_ref, b1a_ref, w1b_ref, b1b_ref, g1_ref, be1_ref,
              rm1_ref, rv1_ref, w2a_ref, z_ref):
  agg = agg_ref[0, :_N, :] + agg_ref[1, :_N, :]
  t = jax.nn.relu(agg + y_ref[...] + b1a_ref[...])
  h = jnp.dot(t, w1b_ref[...], preferred_element_type=jnp.float32)
  h = jax.nn.relu(h + b1b_ref[...])
  h = (h - rm1_ref[...]) / jnp.sqrt(rv1_ref[...] + 1e-5) * g1_ref[...] \
      + be1_ref[...]
  z_ref[...] = jnp.dot(h, w2a_ref[...], preferred_element_type=jnp.float32)


def _tail_body(agg_ref, z_ref, b2a_ref, w2b_ref, b2b_ref, g2_ref, be2_ref,
               rm2_ref, rv2_ref, batch_ref, wf1_ref, bf1_ref, wf2_ref,
               bf2_ref, o_ref):
  agg = agg_ref[0, :_N, :] + agg_ref[1, :_N, :]
  t = jax.nn.relu(agg + z_ref[...] + b2a_ref[...])
  h2 = jnp.dot(t, w2b_ref[...], preferred_element_type=jnp.float32)
  h2 = jax.nn.relu(h2 + b2b_ref[...])
  h2 = (h2 - rm2_ref[...]) / jnp.sqrt(rv2_ref[...] + 1e-5) * g2_ref[...] \
      + be2_ref[...]

  seg = (lax.broadcasted_iota(jnp.int32, (_G, _N), 0)
         == batch_ref[...]).astype(jnp.float32)
  pooled = jnp.dot(seg, h2, preferred_element_type=jnp.float32)
  counts = jnp.maximum(jnp.sum(seg, axis=1, keepdims=True), 1.0)
  pooled = pooled / counts

  h3 = jax.nn.relu(
      jnp.dot(pooled, wf1_ref[...], preferred_element_type=jnp.float32)
      + bf1_ref[...])
  o_ref[...] = jnp.dot(h3, wf2_ref[...],
                       preferred_element_type=jnp.float32) + bf2_ref[...]


def kernel(x, edge_index, batch, W1a, b1a, W1b, b1b, g1, be1, rm1, rv1,
           W2a, b2a, W2b, b2b, g2, be2, rm2, rv2, Wf1, bf1, Wf2, bf2):
  f32 = jnp.float32
  zeros_blk = jnp.zeros((_RPS, _DIM), f32)
  r = lambda v: v.reshape(1, -1)

  y = pl.pallas_call(
      _proj_body,
      out_shape=jax.ShapeDtypeStruct((_N, _DIM), f32),
  )(x, W1a)

  npad_e = _EPAD - _E
  src = jnp.concatenate(
      [edge_index[0], jnp.zeros((npad_e,), jnp.int32)]).reshape(-1, _K)
  dst = jnp.concatenate(
      [edge_index[1], jnp.full((npad_e,), _NPAD - 8, jnp.int32)]
  ).reshape(-1, _K)
  perm = jnp.asarray(_PERM, jnp.int32)
  agg1 = _segsum_sc(y, y.astype(jnp.bfloat16)[:, perm], src, dst, zeros_blk)

  z = pl.pallas_call(
      _mid_body,
      out_shape=jax.ShapeDtypeStruct((_N, _DIM), f32),
  )(agg1, y, r(b1a), W1b, r(b1b), r(g1), r(be1), r(rm1), r(rv1), W2a)

  agg2 = _segsum_sc(z, z.astype(jnp.bfloat16)[:, perm], src, dst, zeros_blk)

  out = pl.pallas_call(
      _tail_body,
      out_shape=jax.ShapeDtypeStruct((_G, _DOUT), f32),
  )(agg2, z, r(b2a), W2b, r(b2b), r(g2), r(be2), r(rm2), r(rv2),
    batch.reshape(1, _N), Wf1, r(bf1), Wf2, r(bf2))

  return out


# speedup vs baseline: 1.2010x; 1.2010x over previous
import functools

import jax
import jax.numpy as jnp
from jax import lax
from jax.experimental import pallas as pl
from jax.experimental.pallas import tpu as pltpu
from jax.experimental.pallas import tpu_sc as plsc

_N = 10000
_E = 320000
_DIN = 128
_DIM = 64
_DOUT = 10
_G = 64

_NC = 2
_NS = 16
_NW = _NC * _NS
_NPAD = 10016
_RPS = _NPAD // _NS
_K = 128
_EPAD = 327680
_EPW = _EPAD // _NW
_CHUNKS = _EPW // _K
_PAT = "HSHSS"
_CONV_SLOT = {b: j for j, b in
              enumerate(b for b, p in enumerate(_PAT) if p == "S")}

_PERM = tuple(
    32 * j + (m // 2 if m % 2 == 0 else 16 + (m - 1) // 2)
    for j in (0, 1) for m in range(32)
)


def _segsum_sc(table, table_bf, packed, zeros_blk):
  mesh = plsc.VectorSubcoreMesh(core_axis_name="c", subcore_axis_name="s")

  @functools.partial(
      pl.kernel,
      out_type=jax.ShapeDtypeStruct((_NC, _NPAD, _DIM), jnp.float32),
      mesh=mesh,
      scratch_types=[
          pltpu.VMEM((_CHUNKS, _K), jnp.int32),
          [pltpu.VMEM((_K,), jnp.int32) for _ in _PAT],
          [pltpu.VMEM((_K,), jnp.int32) for _ in _PAT],
          [pltpu.VMEM((_K, _DIM),
                      jnp.float32 if p == "H" else jnp.bfloat16)
           for p in _PAT],
          [pltpu.VMEM((_K, _DIM), jnp.float32)
           for p in _PAT if p == "S"],
          pltpu.VMEM_SHARED((_NPAD, _DIM), jnp.float32),
          pltpu.VMEM_SHARED((_N, _DIM), jnp.bfloat16),
          [pltpu.SemaphoreType.DMA for _ in _PAT],
      ],
      compiler_params=pltpu.CompilerParams(use_tc_tiling_on_sc=False,
                                           needs_layout_passes=False),
  )
  def k(table_hbm, tbf_hbm, pk_hbm, z_hbm, out_hbm, pidx, sbuf, dbuf,
        rows, conv, accum, tbl, sems):
    c = lax.axis_index("c")
    s = lax.axis_index("s")
    wid = c * _NS + s

    pltpu.sync_copy(z_hbm, accum.at[pl.ds(s * _RPS, _RPS)])
    pltpu.sync_copy(tbf_hbm.at[pl.ds(s * (_N // _NS), _N // _NS)],
                    tbl.at[pl.ds(s * (_N // _NS), _N // _NS)])

    pltpu.sync_copy(pk_hbm.at[pl.ds(wid * _CHUNKS, _CHUNKS)], pidx)
    plsc.subcore_barrier()

    nslot = len(_PAT)

    def unpack(i, b):
      @pl.loop(0, _K, step=16)
      def _(v):
        w = pidx[i, pl.ds(v, 16)]
        sbuf[b][pl.ds(v, 16)] = jnp.bitwise_and(w, jnp.int32(65535))
        dbuf[b][pl.ds(v, 16)] = lax.shift_right_logical(w, 16)

    def gather(b):
      tab = table_hbm if _PAT[b] == "H" else tbl
      return pltpu.make_async_copy(tab.at[sbuf[b]], rows[b], sems[b])

    def scatter_add(b, buf):
      pltpu.sync_copy(buf, accum.at[dbuf[b]], add=True)

    def convert(b):
      cv = conv[_CONV_SLOT[b]]

      @pl.loop(0, _K)
      def _(r):
        for j in (0, 1):
          v = rows[b][r, pl.ds(32 * j, 32)]
          w = plsc.bitcast(v, jnp.int32)
          lo = plsc.bitcast(jnp.left_shift(w, 16), jnp.float32)
          hi = plsc.bitcast(jnp.bitwise_and(w, jnp.int32(-65536)),
                            jnp.float32)
          cv[r, pl.ds(32 * j, 16)] = lo
          cv[r, pl.ds(32 * j + 16, 16)] = hi

    for b in range(nslot):
      unpack(b, b)
      gather(b).start()

    @pl.loop(0, _CHUNKS, step=nslot)
    def _(i):
      for b in range(nslot):
        ic = i + b
        gather(b).wait()
        if _PAT[b] == "S":
          convert(b)
          scatter_add(b, conv[_CONV_SLOT[b]])
        else:
          scatter_add(b, rows[b])

        @pl.when(ic + nslot < _CHUNKS)
        def _():
          unpack(ic + nslot, b)
          gather(b).start()

    plsc.subcore_barrier()
    pltpu.sync_copy(accum.at[pl.ds(s * _RPS, _RPS)],
                    out_hbm.at[c, pl.ds(s * _RPS, _RPS)])

  return k(table, table_bf, packed, zeros_blk)


def _proj_body(x_ref, w_ref, o_ref):
  o_ref[...] = jnp.dot(x_ref[...], w_ref[...],
                       preferred_element_type=jnp.float32)


def _mid_body(agg_ref, y_ref, b1a_ref, w1b_ref, b1b_ref, g1_ref, be1_ref,
              rm1_ref, rv1_ref, w2a_ref, z_ref):
  agg = agg_ref[0, :_N, :] + agg_ref[1, :_N, :]
  t = jax.nn.relu(agg + y_ref[...] + b1a_ref[...])
  h = jnp.dot(t, w1b_ref[...], preferred_element_type=jnp.float32)
  h = jax.nn.relu(h + b1b_ref[...])
  h = (h - rm1_ref[...]) / jnp.sqrt(rv1_ref[...] + 1e-5) * g1_ref[...] \
      + be1_ref[...]
  z_ref[...] = jnp.dot(h, w2a_ref[...], preferred_element_type=jnp.float32)


def _tail_body(agg_ref, z_ref, b2a_ref, w2b_ref, b2b_ref, g2_ref, be2_ref,
               rm2_ref, rv2_ref, batch_ref, wf1_ref, bf1_ref, wf2_ref,
               bf2_ref, o_ref):
  agg = agg_ref[0, :_N, :] + agg_ref[1, :_N, :]
  t = jax.nn.relu(agg + z_ref[...] + b2a_ref[...])
  h2 = jnp.dot(t, w2b_ref[...], preferred_element_type=jnp.float32)
  h2 = jax.nn.relu(h2 + b2b_ref[...])
  h2 = (h2 - rm2_ref[...]) / jnp.sqrt(rv2_ref[...] + 1e-5) * g2_ref[...] \
      + be2_ref[...]

  seg = (lax.broadcasted_iota(jnp.int32, (_G, _N), 0)
         == batch_ref[...]).astype(jnp.float32)
  pooled = jnp.dot(seg, h2, preferred_element_type=jnp.float32)
  counts = jnp.maximum(jnp.sum(seg, axis=1, keepdims=True), 1.0)
  pooled = pooled / counts

  h3 = jax.nn.relu(
      jnp.dot(pooled, wf1_ref[...], preferred_element_type=jnp.float32)
      + bf1_ref[...])
  o_ref[...] = jnp.dot(h3, wf2_ref[...],
                       preferred_element_type=jnp.float32) + bf2_ref[...]


def kernel(x, edge_index, batch, W1a, b1a, W1b, b1b, g1, be1, rm1, rv1,
           W2a, b2a, W2b, b2b, g2, be2, rm2, rv2, Wf1, bf1, Wf2, bf2):
  f32 = jnp.float32
  zeros_blk = jnp.zeros((_RPS, _DIM), f32)
  r = lambda v: v.reshape(1, -1)

  y = pl.pallas_call(
      _proj_body,
      out_shape=jax.ShapeDtypeStruct((_N, _DIM), f32),
  )(x, W1a)

  npad_e = _EPAD - _E
  src = jnp.concatenate([edge_index[0], jnp.zeros((npad_e,), jnp.int32)])
  dst = jnp.concatenate(
      [edge_index[1], jnp.full((npad_e,), _NPAD - 8, jnp.int32)])
  packed = jnp.bitwise_or(src, jnp.left_shift(dst, 16)).reshape(-1, _K)
  perm = jnp.asarray(_PERM, jnp.int32)
  agg1 = _segsum_sc(y, y.astype(jnp.bfloat16)[:, perm], packed, zeros_blk)

  z = pl.pallas_call(
      _mid_body,
      out_shape=jax.ShapeDtypeStruct((_N, _DIM), f32),
  )(agg1, y, r(b1a), W1b, r(b1b), r(g1), r(be1), r(rm1), r(rv1), W2a)

  agg2 = _segsum_sc(z, z.astype(jnp.bfloat16)[:, perm], packed, zeros_blk)

  out = pl.pallas_call(
      _tail_body,
      out_shape=jax.ShapeDtypeStruct((_G, _DOUT), f32),
  )(agg2, z, r(b2a), W2b, r(b2b), r(g2), r(be2), r(rm2), r(rv2),
    batch.reshape(1, _N), Wf1, r(bf1), Wf2, r(bf2))

  return out


# speedup vs baseline: 1.2611x; 1.0500x over previous
import functools

import jax
import jax.numpy as jnp
from jax import lax
from jax.experimental import pallas as pl
from jax.experimental.pallas import tpu as pltpu
from jax.experimental.pallas import tpu_sc as plsc

_N = 10000
_E = 320000
_DIN = 128
_DIM = 64
_DOUT = 10
_G = 64

_NC = 2
_NS = 16
_NW = _NC * _NS
_NPAD = 10016
_RPS = _NPAD // _NS
_K = 128
_EPAD = 327680
_EPW = _EPAD // _NW
_CHUNKS = _EPW // _K
_PAT = "HSHSS"
_CONV_SLOT = {b: j for j, b in
              enumerate(b for b, p in enumerate(_PAT) if p == "S")}

_PERM = tuple(
    32 * j + (m // 2 if m % 2 == 0 else 16 + (m - 1) // 2)
    for j in (0, 1) for m in range(32)
)


def _segsum_sc(table, table_bf, packed, zeros_blk):
  mesh = plsc.VectorSubcoreMesh(core_axis_name="c", subcore_axis_name="s")

  @functools.partial(
      pl.kernel,
      out_type=jax.ShapeDtypeStruct((_NC, _NPAD, _DIM), jnp.float32),
      mesh=mesh,
      scratch_types=[
          pltpu.VMEM((_CHUNKS, _K), jnp.int32),
          [pltpu.VMEM((_K,), jnp.int32) for _ in _PAT],
          [pltpu.VMEM((_K,), jnp.int32) for _ in _PAT],
          [pltpu.VMEM((_K, _DIM),
                      jnp.float32 if p == "H" else jnp.bfloat16)
           for p in _PAT],
          [pltpu.VMEM((_K, _DIM), jnp.float32)
           for p in _PAT if p == "S"],
          pltpu.VMEM_SHARED((_NPAD, _DIM), jnp.float32),
          pltpu.VMEM_SHARED((_N, _DIM), jnp.bfloat16),
          [pltpu.SemaphoreType.DMA for _ in _PAT],
      ],
      compiler_params=pltpu.CompilerParams(use_tc_tiling_on_sc=False,
                                           needs_layout_passes=False),
  )
  def k(table_hbm, tbf_hbm, pk_hbm, z_hbm, out_hbm, pidx, sbuf, dbuf,
        rows, conv, accum, tbl, sems):
    c = lax.axis_index("c")
    s = lax.axis_index("s")
    wid = c * _NS + s

    pltpu.sync_copy(z_hbm, accum.at[pl.ds(s * _RPS, _RPS)])
    pltpu.sync_copy(tbf_hbm.at[pl.ds(s * (_N // _NS), _N // _NS)],
                    tbl.at[pl.ds(s * (_N // _NS), _N // _NS)])

    pltpu.sync_copy(pk_hbm.at[pl.ds(wid * _CHUNKS, _CHUNKS)], pidx)
    plsc.subcore_barrier()

    nslot = len(_PAT)

    def unpack(i, b):
      @pl.loop(0, _K, step=16)
      def _(v):
        w = pidx[i, pl.ds(v, 16)]
        sbuf[b][pl.ds(v, 16)] = jnp.bitwise_and(w, jnp.int32(65535))
        dbuf[b][pl.ds(v, 16)] = lax.shift_right_logical(w, 16)

    def gather(b):
      tab = table_hbm if _PAT[b] == "H" else tbl
      return pltpu.make_async_copy(tab.at[sbuf[b]], rows[b], sems[b])

    def scatter_add(b, buf):
      pltpu.sync_copy(buf, accum.at[dbuf[b]], add=True)

    def convert(b):
      cv = conv[_CONV_SLOT[b]]

      @pl.loop(0, _K)
      def _(r):
        for j in (0, 1):
          v = rows[b][r, pl.ds(32 * j, 32)]
          w = plsc.bitcast(v, jnp.int32)
          lo = plsc.bitcast(jnp.left_shift(w, 16), jnp.float32)
          hi = plsc.bitcast(jnp.bitwise_and(w, jnp.int32(-65536)),
                            jnp.float32)
          cv[r, pl.ds(32 * j, 16)] = lo
          cv[r, pl.ds(32 * j + 16, 16)] = hi

    for b in range(nslot):
      unpack(b, b)
      gather(b).start()

    @pl.loop(0, _CHUNKS, step=nslot)
    def _(i):
      for b in range(nslot):
        ic = i + b
        gather(b).wait()
        if _PAT[b] == "S":
          convert(b)
          scatter_add(b, conv[_CONV_SLOT[b]])
        else:
          scatter_add(b, rows[b])

        @pl.when(ic + nslot < _CHUNKS)
        def _():
          unpack(ic + nslot, b)
          gather(b).start()

    plsc.subcore_barrier()
    pltpu.sync_copy(accum.at[pl.ds(s * _RPS, _RPS)],
                    out_hbm.at[c, pl.ds(s * _RPS, _RPS)])

  return k(table, table_bf, packed, zeros_blk)


_PADVAL = (_NPAD - 8) << 16


def _proj_body(x_ref, w_ref, wp_ref, src_ref, dst_ref, o_ref, ob_ref,
               pk_ref):
  o_ref[...] = jnp.dot(x_ref[...], w_ref[...],
                       preferred_element_type=jnp.float32)
  ob_ref[...] = jnp.dot(x_ref[...], wp_ref[...],
                        preferred_element_type=jnp.float32
                        ).astype(jnp.bfloat16)
  ne = _E // _K
  pk_ref[:ne, :] = jnp.bitwise_or(src_ref[...],
                                  jnp.left_shift(dst_ref[...], 16))
  pk_ref[ne:, :] = jnp.full((_EPAD // _K - ne, _K), _PADVAL, jnp.int32)


def _mid_body(agg_ref, y_ref, b1a_ref, w1b_ref, b1b_ref, g1_ref, be1_ref,
              rm1_ref, rv1_ref, w2a_ref, w2ap_ref, z_ref, zb_ref):
  agg = agg_ref[0, :_N, :] + agg_ref[1, :_N, :]
  t = jax.nn.relu(agg + y_ref[...] + b1a_ref[...])
  h = jnp.dot(t, w1b_ref[...], preferred_element_type=jnp.float32)
  h = jax.nn.relu(h + b1b_ref[...])
  h = (h - rm1_ref[...]) / jnp.sqrt(rv1_ref[...] + 1e-5) * g1_ref[...] \
      + be1_ref[...]
  z_ref[...] = jnp.dot(h, w2a_ref[...], preferred_element_type=jnp.float32)
  zb_ref[...] = jnp.dot(h, w2ap_ref[...],
                        preferred_element_type=jnp.float32
                        ).astype(jnp.bfloat16)


def _tail_body(agg_ref, z_ref, b2a_ref, w2b_ref, b2b_ref, g2_ref, be2_ref,
               rm2_ref, rv2_ref, batch_ref, wf1_ref, bf1_ref, wf2_ref,
               bf2_ref, o_ref):
  agg = agg_ref[0, :_N, :] + agg_ref[1, :_N, :]
  t = jax.nn.relu(agg + z_ref[...] + b2a_ref[...])
  h2 = jnp.dot(t, w2b_ref[...], preferred_element_type=jnp.float32)
  h2 = jax.nn.relu(h2 + b2b_ref[...])
  h2 = (h2 - rm2_ref[...]) / jnp.sqrt(rv2_ref[...] + 1e-5) * g2_ref[...] \
      + be2_ref[...]

  seg = (lax.broadcasted_iota(jnp.int32, (_G, _N), 0)
         == batch_ref[...]).astype(jnp.float32)
  pooled = jnp.dot(seg, h2, preferred_element_type=jnp.float32)
  counts = jnp.maximum(jnp.sum(seg, axis=1, keepdims=True), 1.0)
  pooled = pooled / counts

  h3 = jax.nn.relu(
      jnp.dot(pooled, wf1_ref[...], preferred_element_type=jnp.float32)
      + bf1_ref[...])
  o_ref[...] = jnp.dot(h3, wf2_ref[...],
                       preferred_element_type=jnp.float32) + bf2_ref[...]


def kernel(x, edge_index, batch, W1a, b1a, W1b, b1b, g1, be1, rm1, rv1,
           W2a, b2a, W2b, b2b, g2, be2, rm2, rv2, Wf1, bf1, Wf2, bf2):
  f32 = jnp.float32
  zeros_blk = jnp.zeros((_RPS, _DIM), f32)
  r = lambda v: v.reshape(1, -1)

  perm = jnp.asarray(_PERM, jnp.int32)

  y, ybf, packed = pl.pallas_call(
      _proj_body,
      out_shape=(jax.ShapeDtypeStruct((_N, _DIM), f32),
                 jax.ShapeDtypeStruct((_N, _DIM), jnp.bfloat16),
                 jax.ShapeDtypeStruct((_EPAD // _K, _K), jnp.int32)),
  )(x, W1a, W1a[:, perm], edge_index[0].reshape(-1, _K),
    edge_index[1].reshape(-1, _K))

  agg1 = _segsum_sc(y, ybf, packed, zeros_blk)

  z, zbf = pl.pallas_call(
      _mid_body,
      out_shape=(jax.ShapeDtypeStruct((_N, _DIM), f32),
                 jax.ShapeDtypeStruct((_N, _DIM), jnp.bfloat16)),
  )(agg1, y, r(b1a), W1b, r(b1b), r(g1), r(be1), r(rm1), r(rv1), W2a,
    W2a[:, perm])

  agg2 = _segsum_sc(z, zbf, packed, zeros_blk)

  out = pl.pallas_call(
      _tail_body,
      out_shape=jax.ShapeDtypeStruct((_G, _DOUT), f32),
  )(agg2, z, r(b2a), W2b, r(b2b), r(g2), r(be2), r(rm2), r(rv2),
    batch.reshape(1, _N), Wf1, r(bf1), Wf2, r(bf2))

  return out
